# baseline (speedup 1.0000x reference)
import functools

import jax
import jax.numpy as jnp
from jax import lax
from jax.experimental import pallas as pl
from jax.experimental.pallas import tpu as pltpu
from jax.experimental.pallas import tpu_sc as plsc

_INFO = plsc.get_sparse_core_info()
_NC, _NS, _LANES = _INFO.num_cores, _INFO.num_subcores, _INFO.num_lanes
_NW = _NC * _NS

_HALF = 100


def _build(B, L, E, V, P):
    flat = B * L
    assert flat % (_NW * L) == 0
    chunks = flat // (_NW * L)
    assert L == 2 * _HALF and E % _LANES == 0
    evecs = E // _LANES

    mesh = plsc.VectorSubcoreMesh(core_axis_name="c", subcore_axis_name="s")

    @functools.partial(
        pl.kernel,
        mesh=mesh,
        out_type=jax.ShapeDtypeStruct((flat, E), jnp.float32),
        compiler_params=pltpu.CompilerParams(use_tc_tiling_on_sc=False),
        scratch_types=[
            pltpu.VMEM((2, _HALF), jnp.int32),
            pltpu.VMEM((L, E), jnp.float32),
            pltpu.VMEM((L, E), jnp.float32),
            pltpu.SemaphoreType.DMA,
        ],
    )
    def run(src_hbm, word_hbm, pos_hbm, out_hbm, idx_v, pos_v, rows_v, gsem):
        wid = lax.axis_index("s") * _NC + lax.axis_index("c")
        pltpu.sync_copy(pos_hbm.at[pl.ds(0, L)], pos_v)

        def chunk_body(j, carry):
            row = wid * chunks + j
            pltpu.sync_copy(src_hbm.at[pl.ds(row * 2, 2)], idx_v)
            cp0 = pltpu.async_copy(
                word_hbm.at[idx_v.at[0]], rows_v.at[pl.ds(0, _HALF)], gsem)
            cp1 = pltpu.async_copy(
                word_hbm.at[idx_v.at[1]], rows_v.at[pl.ds(_HALF, _HALF)], gsem)
            cp0.wait()
            cp1.wait()

            def add_body(r, c2):
                for c in range(evecs):
                    x = pos_v[r, pl.ds(c * _LANES, _LANES)]
                    plsc.addupdate(rows_v.at[r, pl.ds(c * _LANES, _LANES)], x)
                return c2

            lax.fori_loop(0, L, add_body, 0)
            pltpu.sync_copy(rows_v, out_hbm.at[pl.ds(row * L, L)])
            return carry

        lax.fori_loop(0, chunks, chunk_body, 0)

    return run


def kernel(src, seg, word_table, pos_table):
    B, L = src.shape
    V, E = word_table.shape
    P = pos_table.shape[0]
    src2 = src.reshape(B * L // _HALF, _HALF).astype(jnp.int32)
    run = _build(B, L, E, V, P)
    out = run(src2, word_table, pos_table)
    return out.reshape(B, L, E)


# speedup vs baseline: 1.2108x; 1.2108x over previous
import functools

import jax
import jax.numpy as jnp
from jax import lax
from jax.experimental import pallas as pl
from jax.experimental.pallas import tpu as pltpu
from jax.experimental.pallas import tpu_sc as plsc

_INFO = plsc.get_sparse_core_info()
_NC, _NS, _LANES = _INFO.num_cores, _INFO.num_subcores, _INFO.num_lanes
_NW = _NC * _NS

_HALF = 100
_NBUF = 4


def _build(B, L, E):
    flat = B * L
    assert flat % (_NW * L) == 0
    chunks = flat // (_NW * L)
    assert chunks % _NBUF == 0
    groups = chunks // _NBUF
    assert L == 2 * _HALF and E % _LANES == 0
    evecs = E // _LANES
    irows = chunks * 2

    mesh = plsc.VectorSubcoreMesh(core_axis_name="c", subcore_axis_name="s")

    @functools.partial(
        pl.kernel,
        mesh=mesh,
        out_type=jax.ShapeDtypeStruct((flat, E), jnp.float32),
        compiler_params=pltpu.CompilerParams(use_tc_tiling_on_sc=False),
        scratch_types=(
            [
                pltpu.VMEM((irows, _HALF), jnp.int32),
                pltpu.VMEM((L, E), jnp.float32),
                pltpu.VMEM((_NBUF, L, E), jnp.float32),
            ]
            + [pltpu.SemaphoreType.DMA] * (2 * _NBUF)
        ),
    )
    def run(src_hbm, word_hbm, pos_hbm, out_hbm, idx_v, pos_v, rows_v, *sems):
        gsems, osems = sems[:_NBUF], sems[_NBUF:]
        wid = lax.axis_index("s") * _NC + lax.axis_index("c")
        pltpu.sync_copy(src_hbm.at[pl.ds(wid * irows, irows)], idx_v)
        pltpu.sync_copy(pos_hbm.at[pl.ds(0, L)], pos_v)
        out_base = wid * chunks

        def issue_gather(j, b):
            pltpu.async_copy(word_hbm.at[idx_v.at[2 * j]],
                             rows_v.at[b, pl.ds(0, _HALF)], gsems[b])
            pltpu.async_copy(word_hbm.at[idx_v.at[2 * j + 1]],
                             rows_v.at[b, pl.ds(_HALF, _HALF)], gsems[b])

        def wait_gather(b):
            pltpu.make_async_copy(word_hbm.at[idx_v.at[0]],
                                  rows_v.at[b, pl.ds(0, _HALF)],
                                  gsems[b]).wait()
            pltpu.make_async_copy(word_hbm.at[idx_v.at[1]],
                                  rows_v.at[b, pl.ds(_HALF, _HALF)],
                                  gsems[b]).wait()

        def wait_out(b):
            pltpu.make_async_copy(rows_v.at[b], out_hbm.at[pl.ds(0, L)],
                                  osems[b]).wait()

        issue_gather(0, 0)
        issue_gather(1, 1)

        def group_body(g, carry):
            for b in range(_NBUF):
                j = g * _NBUF + b
                jn = j + 2
                bn = (b + 2) % _NBUF

                @pl.when(jn < chunks)
                def _prefetch():
                    @pl.when(j >= 2)
                    def _drain():
                        wait_out(bn)
                    issue_gather(jn, bn)

                wait_gather(b)

                def _add(r, c2):
                    for c in range(evecs):
                        x = pos_v[r, pl.ds(c * _LANES, _LANES)]
                        plsc.addupdate(rows_v.at[b, r, pl.ds(c * _LANES, _LANES)], x)
                    return c2

                lax.fori_loop(0, L, _add, 0)

                pltpu.async_copy(rows_v.at[b],
                                 out_hbm.at[pl.ds((out_base + j) * L, L)],
                                 osems[b])
            return carry

        lax.fori_loop(0, groups, group_body, 0)
        for b in range(_NBUF):
            wait_out(b)

    return run


def kernel(src, seg, word_table, pos_table):
    B, L = src.shape
    V, E = word_table.shape
    src2 = src.reshape(B * L // _HALF, _HALF).astype(jnp.int32)
    run = _build(B, L, E)
    out = run(src2, word_table, pos_table)
    return out.reshape(B, L, E)
